# baseline (speedup 1.0000x reference)
import functools

import jax
import jax.numpy as jnp
from jax import lax
from jax.experimental import pallas as pl
from jax.experimental.pallas import tpu as pltpu
from jax.experimental.pallas import tpu_sc as plsc

NUM_PATCHES = 1024
PROJ_DIM = 768
BATCH = 64

NUM_CORES = 2
NUM_SUBCORES = 16
NW = NUM_CORES * NUM_SUBCORES
ROWS_PER_W = NUM_PATCHES // NW
LANES = 16
COL_CHUNKS = PROJ_DIM // LANES
NBUF = 4


def _sc_body(patches_hbm, pos_hbm, out_hbm, pos_v, b0, b1, b2, b3,
             si0, si1, si2, si3, so0, so1, so2, so3):
    wid = lax.axis_index("s") * NUM_CORES + lax.axis_index("c")
    base = wid * ROWS_PER_W
    rows = pl.ds(base, ROWS_PER_W)
    pltpu.sync_copy(pos_hbm.at[rows], pos_v)

    bufs = [b0, b1, b2, b3]
    sis = [si0, si1, si2, si3]
    sos = [so0, so1, so2, so3]

    for b in range(NBUF - 1):
        pltpu.async_copy(patches_hbm.at[b, rows], bufs[b], sis[b])

    @pl.loop(0, BATCH, step=NBUF)
    def _batch(b):
        for k in range(NBUF):
            i = b + k
            t = (k + NBUF - 1) % NBUF
            pltpu.make_async_copy(patches_hbm.at[i, rows], bufs[k],
                                  sis[k]).wait()

            @pl.loop(0, ROWS_PER_W)
            def _row(r):
                for c in range(COL_CHUNKS):
                    sl = pl.ds(c * LANES, LANES)
                    plsc.addupdate(bufs[k].at[r, sl], pos_v[r, sl])

            pltpu.async_copy(bufs[k], out_hbm.at[i, rows], sos[k])

            @pl.when(i + NBUF - 1 < BATCH)
            def _next_load():
                @pl.when(i >= 1)
                def _wait_prev_store():
                    pltpu.make_async_copy(bufs[t], out_hbm.at[i - 1, rows],
                                          sos[t]).wait()

                pltpu.async_copy(patches_hbm.at[i + NBUF - 1, rows], bufs[t],
                                 sis[t])

    for k in range(NBUF):
        pltpu.make_async_copy(bufs[k], out_hbm.at[BATCH - NBUF + k, rows],
                              sos[k]).wait()


_sc_kernel = functools.partial(
    pl.kernel,
    out_type=jax.ShapeDtypeStruct((BATCH, NUM_PATCHES, PROJ_DIM), jnp.float32),
    mesh=plsc.VectorSubcoreMesh(core_axis_name="c", subcore_axis_name="s"),
    scratch_types=[
        pltpu.VMEM((ROWS_PER_W, PROJ_DIM), jnp.float32),
        pltpu.VMEM((ROWS_PER_W, PROJ_DIM), jnp.float32),
        pltpu.VMEM((ROWS_PER_W, PROJ_DIM), jnp.float32),
        pltpu.VMEM((ROWS_PER_W, PROJ_DIM), jnp.float32),
        pltpu.VMEM((ROWS_PER_W, PROJ_DIM), jnp.float32),
        pltpu.SemaphoreType.DMA,
        pltpu.SemaphoreType.DMA,
        pltpu.SemaphoreType.DMA,
        pltpu.SemaphoreType.DMA,
        pltpu.SemaphoreType.DMA,
        pltpu.SemaphoreType.DMA,
        pltpu.SemaphoreType.DMA,
        pltpu.SemaphoreType.DMA,
    ],
)(_sc_body)


def kernel(patches, pos_table):
    return _sc_kernel(patches, pos_table)


# speedup vs baseline: 1.1217x; 1.1217x over previous
import functools

import jax
import jax.numpy as jnp
from jax import lax
from jax.experimental import pallas as pl
from jax.experimental.pallas import tpu as pltpu
from jax.experimental.pallas import tpu_sc as plsc

NUM_PATCHES = 1024
PROJ_DIM = 768
BATCH = 64

NUM_CORES = 2
NUM_SUBCORES = 16
NW = NUM_CORES * NUM_SUBCORES
ROWS_PER_W = NUM_PATCHES // NW
LANES = 16

B_BLK = 4


def _gather_body(pos_hbm, out_hbm, idx_v, rows_v, sem):
    wid = lax.axis_index("s") * NUM_CORES + lax.axis_index("c")
    base = wid * ROWS_PER_W
    for c in range(ROWS_PER_W // LANES):
        idx_v[pl.ds(c * LANES, LANES)] = base + c * LANES + lax.iota(
            jnp.int32, LANES)
    pltpu.async_copy(pos_hbm.at[idx_v], rows_v, sem).wait()
    pltpu.sync_copy(rows_v, out_hbm.at[pl.ds(base, ROWS_PER_W)])


_sc_gather = functools.partial(
    pl.kernel,
    out_type=jax.ShapeDtypeStruct((NUM_PATCHES, PROJ_DIM), jnp.float32),
    mesh=plsc.VectorSubcoreMesh(core_axis_name="c", subcore_axis_name="s"),
    scratch_types=[
        pltpu.VMEM((ROWS_PER_W,), jnp.int32),
        pltpu.VMEM((ROWS_PER_W, PROJ_DIM), jnp.float32),
        pltpu.SemaphoreType.DMA,
    ],
)(_gather_body)


def _add_body(patches_ref, pos_ref, out_ref):
    out_ref[...] = patches_ref[...] + pos_ref[...][None]


def _tc_add(patches, pos_emb):
    return pl.pallas_call(
        _add_body,
        grid=(BATCH // B_BLK,),
        in_specs=[
            pl.BlockSpec((B_BLK, NUM_PATCHES, PROJ_DIM), lambda b: (b, 0, 0)),
            pl.BlockSpec((NUM_PATCHES, PROJ_DIM), lambda b: (0, 0)),
        ],
        out_specs=pl.BlockSpec((B_BLK, NUM_PATCHES, PROJ_DIM),
                               lambda b: (b, 0, 0)),
        out_shape=jax.ShapeDtypeStruct((BATCH, NUM_PATCHES, PROJ_DIM),
                                       jnp.float32),
    )(patches, pos_emb)


def kernel(patches, pos_table):
    pos_emb = _sc_gather(pos_table)
    return _tc_add(patches, pos_emb)


# speedup vs baseline: 1.3035x; 1.1621x over previous
import jax
import jax.numpy as jnp
from jax.experimental import pallas as pl
from jax.experimental.pallas import tpu as pltpu

NUM_PATCHES = 1024
PROJ_DIM = 768
BATCH = 64

CB = 2
NCH = BATCH // CB
K = 4


def _body(patches_hbm, pos_ref, out_hbm, b0, b1, b2, b3,
          si0, si1, si2, si3, so0, so1, so2, so3):
    bufs = [b0, b1, b2, b3]
    sis = [si0, si1, si2, si3]
    sos = [so0, so1, so2, so3]

    def chunk(c):
        return pl.ds(c * CB, CB)

    for j in range(K - 1):
        pltpu.make_async_copy(patches_hbm.at[chunk(j)], bufs[j], sis[j]).start()

    for c in range(NCH):
        k = c % K
        pltpu.make_async_copy(patches_hbm.at[chunk(c)], bufs[k], sis[k]).wait()
        bufs[k][...] = bufs[k][...] + pos_ref[...][None]
        pltpu.make_async_copy(bufs[k], out_hbm.at[chunk(c)], sos[k]).start()
        n = c + K - 1
        if n < NCH:
            t = n % K
            if c >= 1:
                pltpu.make_async_copy(bufs[t], out_hbm.at[chunk(c - 1)],
                                      sos[t]).wait()
            pltpu.make_async_copy(patches_hbm.at[chunk(n)], bufs[t],
                                  sis[t]).start()

    for j in range(K):
        c = NCH - K + j
        pltpu.make_async_copy(bufs[c % K], out_hbm.at[chunk(c)],
                              sos[c % K]).wait()


def kernel(patches, pos_table):
    return pl.pallas_call(
        _body,
        in_specs=[
            pl.BlockSpec(memory_space=pltpu.MemorySpace.HBM),
            pl.BlockSpec(memory_space=pltpu.MemorySpace.VMEM),
        ],
        out_specs=pl.BlockSpec(memory_space=pltpu.MemorySpace.HBM),
        out_shape=jax.ShapeDtypeStruct((BATCH, NUM_PATCHES, PROJ_DIM),
                                       jnp.float32),
        scratch_shapes=[
            pltpu.VMEM((CB, NUM_PATCHES, PROJ_DIM), jnp.float32),
            pltpu.VMEM((CB, NUM_PATCHES, PROJ_DIM), jnp.float32),
            pltpu.VMEM((CB, NUM_PATCHES, PROJ_DIM), jnp.float32),
            pltpu.VMEM((CB, NUM_PATCHES, PROJ_DIM), jnp.float32),
            pltpu.SemaphoreType.DMA,
            pltpu.SemaphoreType.DMA,
            pltpu.SemaphoreType.DMA,
            pltpu.SemaphoreType.DMA,
            pltpu.SemaphoreType.DMA,
            pltpu.SemaphoreType.DMA,
            pltpu.SemaphoreType.DMA,
            pltpu.SemaphoreType.DMA,
        ],
    )(patches, pos_table)


# speedup vs baseline: 1.3124x; 1.0069x over previous
import jax
import jax.numpy as jnp
from jax.experimental import pallas as pl
from jax.experimental.pallas import tpu as pltpu

NUM_PATCHES = 1024
PROJ_DIM = 768
BATCH = 64

SIZES = [1, 1, 2] + [4] * 14 + [2, 1, 1]
STARTS = []
_s = 0
for _sz in SIZES:
    STARTS.append(_s)
    _s += _sz
assert _s == BATCH
NCH = len(SIZES)
MAXCB = max(SIZES)
K = 4


def _body(patches_hbm, pos_ref, out_hbm, b0, b1, b2, b3,
          si0, si1, si2, si3, so0, so1, so2, so3):
    bufs = [b0, b1, b2, b3]
    sis = [si0, si1, si2, si3]
    sos = [so0, so1, so2, so3]

    def in_copy(c, k):
        return pltpu.make_async_copy(
            patches_hbm.at[pl.ds(STARTS[c], SIZES[c])],
            bufs[k].at[pl.ds(0, SIZES[c])], sis[k])

    def out_copy(c, k):
        return pltpu.make_async_copy(
            bufs[k].at[pl.ds(0, SIZES[c])],
            out_hbm.at[pl.ds(STARTS[c], SIZES[c])], sos[k])

    for j in range(K - 1):
        in_copy(j, j).start()

    for c in range(NCH):
        k = c % K
        in_copy(c, k).wait()
        sl = pl.ds(0, SIZES[c])
        bufs[k][sl] = bufs[k][sl] + pos_ref[...][None]
        out_copy(c, k).start()
        n = c + K - 1
        if n < NCH:
            t = n % K
            if c >= 1:
                out_copy(c - 1, t).wait()
            in_copy(n, t).start()

    for c in range(NCH - K, NCH):
        out_copy(c, c % K).wait()


def kernel(patches, pos_table):
    return pl.pallas_call(
        _body,
        in_specs=[
            pl.BlockSpec(memory_space=pltpu.MemorySpace.HBM),
            pl.BlockSpec(memory_space=pltpu.MemorySpace.VMEM),
        ],
        out_specs=pl.BlockSpec(memory_space=pltpu.MemorySpace.HBM),
        out_shape=jax.ShapeDtypeStruct((BATCH, NUM_PATCHES, PROJ_DIM),
                                       jnp.float32),
        scratch_shapes=[
            pltpu.VMEM((MAXCB, NUM_PATCHES, PROJ_DIM), jnp.float32),
            pltpu.VMEM((MAXCB, NUM_PATCHES, PROJ_DIM), jnp.float32),
            pltpu.VMEM((MAXCB, NUM_PATCHES, PROJ_DIM), jnp.float32),
            pltpu.VMEM((MAXCB, NUM_PATCHES, PROJ_DIM), jnp.float32),
            pltpu.SemaphoreType.DMA,
            pltpu.SemaphoreType.DMA,
            pltpu.SemaphoreType.DMA,
            pltpu.SemaphoreType.DMA,
            pltpu.SemaphoreType.DMA,
            pltpu.SemaphoreType.DMA,
            pltpu.SemaphoreType.DMA,
            pltpu.SemaphoreType.DMA,
        ],
    )(patches, pos_table)


# speedup vs baseline: 1.3173x; 1.0037x over previous
import jax
import jax.numpy as jnp
from jax.experimental import pallas as pl

NUM_PATCHES = 1024
PROJ_DIM = 768
BATCH = 64

B_BLK = 4


def _add_body(patches_ref, pos_ref, out_ref):
    out_ref[...] = patches_ref[...] + pos_ref[...][None]


def kernel(patches, pos_table):
    return pl.pallas_call(
        _add_body,
        grid=(BATCH // B_BLK,),
        in_specs=[
            pl.BlockSpec((B_BLK, NUM_PATCHES, PROJ_DIM), lambda b: (b, 0, 0)),
            pl.BlockSpec((NUM_PATCHES, PROJ_DIM), lambda b: (0, 0)),
        ],
        out_specs=pl.BlockSpec((B_BLK, NUM_PATCHES, PROJ_DIM), lambda b: (b, 0, 0)),
        out_shape=jax.ShapeDtypeStruct((BATCH, NUM_PATCHES, PROJ_DIM), jnp.float32),
    )(patches, pos_table)
